# Initial kernel scaffold; baseline (speedup 1.0000x reference)
import jax
import jax.numpy as jnp
from jax.experimental import pallas as pl


def kernel(x, edge_index, batch, graph_attr, W_l1, b_l1, W_r1, W_l2, b_l2, W_r2, W1, b1, W2, b2):
    raise NotImplementedError("write your pallas kernel here")


# speedup vs baseline: 5.4895x; 5.4895x over previous
import functools

import jax
import jax.numpy as jnp
from jax import lax
from jax.experimental import pallas as pl
from jax.experimental.pallas import tpu as pltpu
from jax.experimental.pallas import tpu_sc as plsc

_N = 10000
_E = 320000
_F = 128
_H = 256
_G = 64
_A = 16
_OUT = 8

_NPAD = 10240
_EPAD = 327680
_CHUNK = 128
_CFLAT = 65 * _NPAD

@functools.cache
def _make_sc_degree_and_c():
  deco = functools.partial(
      pl.kernel,
      mesh=plsc.VectorSubcoreMesh(core_axis_name="c", subcore_axis_name="s"),
      compiler_params=pltpu.CompilerParams(needs_layout_passes=False),
      out_type=(
          jax.ShapeDtypeStruct((_NPAD,), jnp.float32),
          jax.ShapeDtypeStruct((2, _G * _NPAD), jnp.float32),
      ),
      scratch_types=[
          pltpu.VMEM((4160,), jnp.float32),
          pltpu.VMEM((_CHUNK,), jnp.float32),
          pltpu.VMEM((_CHUNK,), jnp.int32),
          pltpu.VMEM((_CHUNK,), jnp.int32),
          pltpu.VMEM((_CHUNK,), jnp.int32),
          pltpu.VMEM((_CHUNK,), jnp.float32),
          pltpu.VMEM((640,), jnp.float32),
          pltpu.VMEM((640,), jnp.float32),
          pltpu.VMEM((_NPAD,), jnp.int32),
          pltpu.VMEM((_NPAD,), jnp.float32),
          pltpu.VMEM_SHARED((_NPAD,), jnp.float32),
          pltpu.VMEM_SHARED((_NPAD,), jnp.float32),
          pltpu.VMEM_SHARED((_CFLAT,), jnp.float32),
      ],
  )
  return deco(_sc_degree_and_c_body)


def _sc_degree_and_c_body(src_hbm, dst_hbm, batch_hbm, inv_out, c_out,
                          zbuf, ones_b, idxb, db, sb, updb, dslice, islice,
                          batch_v, inv_v, deg_s, inv_s, c_s):
    c = lax.axis_index("c")
    s = lax.axis_index("s")

    def _fill(i, _):
        zbuf[pl.ds(pl.multiple_of(i * 16, 16), 16)] = jnp.zeros((16,), jnp.float32)
        return 0
    lax.fori_loop(0, 260, _fill, 0)
    for i in range(8):
        ones_b[pl.ds(i * 16, 16)] = jnp.ones((16,), jnp.float32)

    pltpu.sync_copy(zbuf.at[pl.ds(0, 640)],
                    deg_s.at[pl.ds(pl.multiple_of(s * 640, 640), 640)])
    cz_base = pl.multiple_of(s * 41600, 8)

    def _zero_c(j, _):
        pltpu.sync_copy(zbuf, c_s.at[pl.ds(pl.multiple_of(cz_base + j * 4160, 8), 4160)])
        return 0
    lax.fori_loop(0, 10, _zero_c, 0)
    plsc.subcore_barrier()

    deg_base = pl.multiple_of(s * 20480, 128)

    def _deg(j, _):
        b = pl.multiple_of(deg_base + j * _CHUNK, _CHUNK)
        pltpu.sync_copy(dst_hbm.at[pl.ds(b, _CHUNK)], idxb)
        pltpu.sync_copy(ones_b, deg_s.at[idxb], add=True)
        return 0
    lax.fori_loop(0, 160, _deg, 0)
    plsc.subcore_barrier()

    off = pl.multiple_of(s * 640, 640)
    pltpu.sync_copy(deg_s.at[pl.ds(off, 640)], dslice)

    def _inv(k, _):
        d16 = dslice[pl.ds(pl.multiple_of(k * 16, 16), 16)]
        islice[pl.ds(pl.multiple_of(k * 16, 16), 16)] = 1.0 / jnp.maximum(d16, 1.0)
        return 0
    lax.fori_loop(0, 40, _inv, 0)
    pltpu.sync_copy(islice, inv_s.at[pl.ds(off, 640)])

    @pl.when(c == 0)
    def _():
        pltpu.sync_copy(islice, inv_out.at[pl.ds(off, 640)])
    plsc.subcore_barrier()

    pltpu.sync_copy(batch_hbm, batch_v)
    pltpu.sync_copy(inv_s, inv_v)

    c_base = pl.multiple_of(c * 163840 + s * 10240, 128)

    def _cbuild(j, _):
        b = pl.multiple_of(c_base + j * _CHUNK, _CHUNK)
        pltpu.sync_copy(src_hbm.at[pl.ds(b, _CHUNK)], sb)
        pltpu.sync_copy(dst_hbm.at[pl.ds(b, _CHUNK)], db)
        for k in range(_CHUNK // 16):
            d16 = db[pl.ds(k * 16, 16)]
            s16 = sb[pl.ds(k * 16, 16)]
            t16 = plsc.load_gather(batch_v, [d16])
            w16 = plsc.load_gather(inv_v, [d16])
            idxb[pl.ds(k * 16, 16)] = t16 * _NPAD + s16
            updb[pl.ds(k * 16, 16)] = w16
        pltpu.sync_copy(updb, c_s.at[idxb], add=True)
        return 0
    lax.fori_loop(0, 80, _cbuild, 0)
    plsc.subcore_barrier()

    out_off = pl.multiple_of(s * 40960, 8)
    pltpu.sync_copy(c_s.at[pl.ds(out_off, 40960)], c_out.at[c, pl.ds(out_off, 40960)])


@functools.cache
def _make_sc_aggregate():
  deco = functools.partial(
      pl.kernel,
      mesh=plsc.VectorSubcoreMesh(core_axis_name="c", subcore_axis_name="s"),
      compiler_params=pltpu.CompilerParams(needs_layout_passes=False),
      out_type=jax.ShapeDtypeStruct((2, _NPAD, _F), jnp.float32),
      scratch_types=[
          pltpu.VMEM((_CHUNK,), jnp.int32),
          pltpu.VMEM((_CHUNK,), jnp.int32),
          pltpu.VMEM((_CHUNK, _F), jnp.float32),
          pltpu.VMEM_SHARED((_NPAD, _F), jnp.float32),
          pltpu.SemaphoreType.DMA,
      ],
  )
  return deco(_sc_aggregate_body)


def _sc_aggregate_body(src_hbm, dst_hbm, x_hbm, out_hbm, sidx, didx, rows, agg_s, sem):
    c = lax.axis_index("c")
    s = lax.axis_index("s")

    def _zero(k, _):
        i = k // 8
        j = k % 8
        rows[i, pl.ds(pl.multiple_of(j * 16, 16), 16)] = jnp.zeros((16,), jnp.float32)
        return 0
    lax.fori_loop(0, _CHUNK * 8, _zero, 0)

    row_base = pl.multiple_of(s * 640, 128)

    def _zero_agg(j, _):
        pltpu.sync_copy(rows, agg_s.at[pl.ds(pl.multiple_of(row_base + j * _CHUNK, _CHUNK), _CHUNK)])
        return 0
    lax.fori_loop(0, 5, _zero_agg, 0)
    plsc.subcore_barrier()

    base = pl.multiple_of(c * 163840 + s * 10240, 128)

    def _edge_chunk(j, _):
        b = pl.multiple_of(base + j * _CHUNK, _CHUNK)
        pltpu.sync_copy(src_hbm.at[pl.ds(b, _CHUNK)], sidx)
        pltpu.sync_copy(dst_hbm.at[pl.ds(b, _CHUNK)], didx)
        pltpu.async_copy(x_hbm.at[sidx], rows, sem).wait()
        pltpu.sync_copy(rows, agg_s.at[didx], add=True)
        return 0
    lax.fori_loop(0, 80, _edge_chunk, 0)
    plsc.subcore_barrier()

    pltpu.sync_copy(agg_s.at[pl.ds(pl.multiple_of(s * 640, 640), 640)],
                    out_hbm.at[c, pl.ds(pl.multiple_of(s * 640, 640), 640)])


def _tc_body(x_ref, agg_ref, invd_ref, batch_ref, c2_ref, ga_ref,
             wl1, wr1, wl2, wr2, w1g, w1a, w2p, bl1, bl2, b1r, b2r,
             out_ref, acc_c, acc_q, cnt):
    i = pl.program_id(0)

    @pl.when(i == 0)
    def _():
        acc_c[...] = jnp.zeros_like(acc_c)
        acc_q[...] = jnp.zeros_like(acc_q)
        cnt[...] = jnp.zeros_like(cnt)

    f32 = jnp.float32
    dot = functools.partial(lax.dot_general, preferred_element_type=f32)

    aggsum = agg_ref[0] + agg_ref[1]
    invd = invd_ref[0, 0, :]
    h1 = jnp.maximum(
        dot(aggsum * invd[:, None], wl1[...], (((1,), (1,)), ((), ())))
        + bl1[...]
        + dot(x_ref[...], wr1[...], (((1,), (1,)), ((), ()))),
        0.0)

    cblk = c2_ref[0] + c2_ref[1]
    acc_c[...] += dot(cblk, h1, (((1,), (0,)), ((), ())))

    b = batch_ref[0, 0, :]
    q = (lax.broadcasted_iota(jnp.int32, (_G, 512), 0) == b[None, :]).astype(f32)
    acc_q[...] += dot(q, h1, (((1,), (0,)), ((), ())))
    cnt[...] += jnp.broadcast_to(jnp.sum(q, axis=1)[:, None], (_G, 128))

    @pl.when(i == pl.num_programs(0) - 1)
    def _():
        cvec = cnt[...][:, :1]
        inv_cnt = 1.0 / jnp.maximum(cvec, 1.0)
        nonz = (cvec > 0.0).astype(f32)
        g2 = (dot(acc_c[...] * inv_cnt, wl2[...], (((1,), (1,)), ((), ())))
              + bl2[...] * nonz
              + dot(acc_q[...] * inv_cnt, wr2[...], (((1,), (1,)), ((), ()))))
        zz = jnp.maximum(
            dot(g2, w1g[...], (((1,), (1,)), ((), ())))
            + dot(ga_ref[...], w1a[...], (((1,), (1,)), ((), ())))
            + b1r[...], 0.0)
        out_ref[...] = dot(zz, w2p[...], (((1,), (1,)), ((), ()))) + b2r[...]


def _tc_dense(x_p, aggp, invd3, batch3, c2, ga_p,
              wl1, wr1, wl2, wr2, w1g, w1a, w2p, bl1, bl2, b1r, b2r):
    nblk = _NPAD // 512
    full = lambda shape: pl.BlockSpec(shape, lambda i: tuple(0 for _ in shape))
    return pl.pallas_call(
        _tc_body,
        grid=(nblk,),
        in_specs=[
            pl.BlockSpec((512, _F), lambda i: (i, 0)),
            pl.BlockSpec((2, 512, _F), lambda i: (0, i, 0)),
            pl.BlockSpec((1, 1, 512), lambda i: (i, 0, 0)),
            pl.BlockSpec((1, 1, 512), lambda i: (i, 0, 0)),
            pl.BlockSpec((2, _G, 512), lambda i: (0, 0, i)),
            full((_G, 128)),
            full((_H, _F)), full((_H, _F)), full((_H, _H)), full((_H, _H)),
            full((_H, _H)), full((_H, 128)), full((128, _H)),
            full((1, _H)), full((1, _H)), full((1, _H)), full((1, 128)),
        ],
        out_specs=pl.BlockSpec((_G, 128), lambda i: (0, 0)),
        out_shape=jax.ShapeDtypeStruct((_G, 128), jnp.float32),
        scratch_shapes=[
            pltpu.VMEM((_G, _H), jnp.float32),
            pltpu.VMEM((_G, _H), jnp.float32),
            pltpu.VMEM((_G, 128), jnp.float32),
        ],
    )(x_p, aggp, invd3, batch3, c2, ga_p,
      wl1, wr1, wl2, wr2, w1g, w1a, w2p, bl1, bl2, b1r, b2r)


def kernel(x, edge_index, batch, graph_attr,
           W_l1, b_l1, W_r1, W_l2, b_l2, W_r2, W1, b1, W2, b2):
    f32 = jnp.float32
    src = edge_index[0]
    dst = edge_index[1]
    epad = _EPAD - _E
    src_p = jnp.concatenate([src, jnp.zeros((epad,), jnp.int32)])
    dst_p = jnp.concatenate([dst, jnp.full((epad,), _N, jnp.int32)])
    batch_p = jnp.concatenate([batch, jnp.full((_NPAD - _N,), _G, jnp.int32)])
    x_p = jnp.concatenate([x, jnp.zeros((_NPAD - _N, _F), f32)], axis=0)
    ga_p = jnp.concatenate([graph_attr, jnp.zeros((_G, 128 - _A), f32)], axis=1)

    inv_deg, c2_flat = _make_sc_degree_and_c()(src_p, dst_p, batch_p)
    aggp = _make_sc_aggregate()(src_p, dst_p, x)

    c2 = c2_flat.reshape(2, _G, _NPAD)
    invd3 = inv_deg.reshape(_NPAD // 512, 1, 512)
    batch3 = batch_p.reshape(_NPAD // 512, 1, 512)

    w1g = W1[:, :_H]
    w1a = jnp.concatenate([W1[:, _H:], jnp.zeros((_H, 128 - _A), f32)], axis=1)
    w2p = jnp.concatenate([W2, jnp.zeros((128 - _OUT, _H), f32)], axis=0)
    bl1 = b_l1.reshape(1, _H)
    bl2 = b_l2.reshape(1, _H)
    b1r = b1.reshape(1, _H)
    b2r = jnp.concatenate([b2, jnp.zeros((128 - _OUT,), f32)]).reshape(1, 128)

    out = _tc_dense(x_p, aggp, invd3, batch3, c2, ga_p,
                    W_l1, W_r1, W_l2, W_r2, w1g, w1a, w2p, bl1, bl2, b1r, b2r)
    return out[:, :_OUT]


# speedup vs baseline: 18.0190x; 3.2824x over previous
import functools

import jax
import jax.numpy as jnp
from jax import lax
from jax.experimental import pallas as pl
from jax.experimental.pallas import tpu as pltpu
from jax.experimental.pallas import tpu_sc as plsc

_N = 10000
_E = 320000
_F = 128
_H = 256
_G = 64
_A = 16
_OUT = 8

_NPAD = 10240
_EPAD = 327680
_CHUNK = 128
_CFLAT = 65 * _NPAD

@functools.cache
def _make_sc_degree_and_c():
  deco = functools.partial(
      pl.kernel,
      mesh=plsc.VectorSubcoreMesh(core_axis_name="c", subcore_axis_name="s"),
      compiler_params=pltpu.CompilerParams(needs_layout_passes=False),
      out_type=(
          jax.ShapeDtypeStruct((_NPAD,), jnp.float32),
          jax.ShapeDtypeStruct((2, _G * _NPAD), jnp.float32),
      ),
      scratch_types=[
          pltpu.VMEM((4160,), jnp.float32),
          pltpu.VMEM((_CHUNK,), jnp.float32),
          pltpu.VMEM((_CHUNK,), jnp.int32),
          pltpu.VMEM((160, _CHUNK), jnp.int32),
          pltpu.VMEM((80, _CHUNK), jnp.int32),
          pltpu.VMEM((80, _CHUNK), jnp.int32),
          pltpu.VMEM((_CHUNK,), jnp.float32),
          pltpu.VMEM((640,), jnp.float32),
          pltpu.VMEM((640,), jnp.float32),
          pltpu.VMEM((_NPAD,), jnp.int32),
          pltpu.VMEM((_NPAD,), jnp.float32),
          pltpu.VMEM_SHARED((_NPAD,), jnp.float32),
          pltpu.VMEM_SHARED((_NPAD,), jnp.float32),
          pltpu.VMEM_SHARED((_CFLAT,), jnp.float32),
      ],
  )
  return deco(_sc_degree_and_c_body)


def _sc_degree_and_c_body(src_hbm, dst_hbm, batch_hbm, inv_out, c_out,
                          zbuf, ones_b, idxb, dbig, dc, sc_, updb, dslice,
                          islice, batch_v, inv_v, deg_s, inv_s, c_s):
    c = lax.axis_index("c")
    s = lax.axis_index("s")

    def _fill(i, _):
        zbuf[pl.ds(pl.multiple_of(i * 16, 16), 16)] = jnp.zeros((16,), jnp.float32)
        return 0
    lax.fori_loop(0, 260, _fill, 0)
    for i in range(8):
        ones_b[pl.ds(i * 16, 16)] = jnp.ones((16,), jnp.float32)

    pltpu.sync_copy(zbuf.at[pl.ds(0, 640)],
                    deg_s.at[pl.ds(pl.multiple_of(s * 640, 640), 640)])
    cz_base = pl.multiple_of(s * 41600, 8)

    def _zero_c(j, _):
        pltpu.sync_copy(zbuf, c_s.at[pl.ds(pl.multiple_of(cz_base + j * 4160, 8), 4160)])
        return 0
    lax.fori_loop(0, 10, _zero_c, 0)
    plsc.subcore_barrier()

    pltpu.sync_copy(dst_hbm.at[pl.ds(pl.multiple_of(s * 160, 8), 160)], dbig)

    def _deg(j, _):
        pltpu.sync_copy(ones_b, deg_s.at[dbig.at[j]], add=True)
        return 0
    lax.fori_loop(0, 160, _deg, 0)
    plsc.subcore_barrier()

    off = pl.multiple_of(s * 640, 640)
    pltpu.sync_copy(deg_s.at[pl.ds(off, 640)], dslice)

    def _inv(k, _):
        d16 = dslice[pl.ds(pl.multiple_of(k * 16, 16), 16)]
        islice[pl.ds(pl.multiple_of(k * 16, 16), 16)] = 1.0 / jnp.maximum(d16, 1.0)
        return 0
    lax.fori_loop(0, 40, _inv, 0)
    pltpu.sync_copy(islice, inv_s.at[pl.ds(off, 640)])

    @pl.when(c == 0)
    def _():
        pltpu.sync_copy(islice, inv_out.at[pl.ds(off, 640)])
    plsc.subcore_barrier()

    pltpu.sync_copy(batch_hbm, batch_v)
    pltpu.sync_copy(inv_s, inv_v)

    c_row0 = pl.multiple_of(c * 1280 + s * 80, 8)
    pltpu.sync_copy(src_hbm.at[pl.ds(c_row0, 80)], sc_)
    pltpu.sync_copy(dst_hbm.at[pl.ds(c_row0, 80)], dc)

    def _cbuild(j, _):
        for k in range(_CHUNK // 16):
            d16 = dc[j, pl.ds(k * 16, 16)]
            s16 = sc_[j, pl.ds(k * 16, 16)]
            t16 = plsc.load_gather(batch_v, [d16])
            w16 = plsc.load_gather(inv_v, [d16])
            idxb[pl.ds(k * 16, 16)] = t16 * _NPAD + s16
            updb[pl.ds(k * 16, 16)] = w16
        pltpu.sync_copy(updb, c_s.at[idxb], add=True)
        return 0
    lax.fori_loop(0, 80, _cbuild, 0)
    plsc.subcore_barrier()

    out_off = pl.multiple_of(s * 40960, 8)
    pltpu.sync_copy(c_s.at[pl.ds(out_off, 40960)], c_out.at[c, pl.ds(out_off, 40960)])


@functools.cache
def _make_sc_aggregate():
  deco = functools.partial(
      pl.kernel,
      mesh=plsc.VectorSubcoreMesh(core_axis_name="c", subcore_axis_name="s"),
      compiler_params=pltpu.CompilerParams(needs_layout_passes=False),
      out_type=jax.ShapeDtypeStruct((2, _NPAD, _F), jnp.float32),
      scratch_types=[
          pltpu.VMEM((_CHUNK,), jnp.int32),
          pltpu.VMEM((_CHUNK,), jnp.int32),
          pltpu.VMEM((_CHUNK,), jnp.int32),
          pltpu.VMEM((_CHUNK,), jnp.int32),
          pltpu.VMEM((_CHUNK, _F), jnp.float32),
          pltpu.VMEM((_CHUNK, _F), jnp.float32),
          pltpu.VMEM_SHARED((_NPAD, _F), jnp.float32),
          pltpu.SemaphoreType.DMA,
          pltpu.SemaphoreType.DMA,
      ],
  )
  return deco(_sc_aggregate_body)


def _sc_aggregate_body(src_hbm, dst_hbm, x_hbm, out_hbm,
                       sidx0, didx0, sidx1, didx1, rows0, rows1,
                       agg_s, sem0, sem1):
    c = lax.axis_index("c")
    s = lax.axis_index("s")

    def _zero(k, _):
        i = k // 8
        j = k % 8
        rows0[i, pl.ds(pl.multiple_of(j * 16, 16), 16)] = jnp.zeros((16,), jnp.float32)
        return 0
    lax.fori_loop(0, _CHUNK * 8, _zero, 0)

    row_base = pl.multiple_of(s * 640, 128)

    def _zero_agg(j, _):
        pltpu.sync_copy(rows0, agg_s.at[pl.ds(pl.multiple_of(row_base + j * _CHUNK, _CHUNK), _CHUNK)])
        return 0
    lax.fori_loop(0, 5, _zero_agg, 0)
    plsc.subcore_barrier()

    row0 = c * 1280 + s * 80

    pltpu.sync_copy(src_hbm.at[row0], sidx0)
    pltpu.sync_copy(dst_hbm.at[row0], didx0)
    pltpu.async_copy(x_hbm.at[sidx0], rows0, sem0)

    def _pair(jj, _):
        r0 = row0 + jj * 2
        pltpu.sync_copy(src_hbm.at[r0 + 1], sidx1)
        pltpu.sync_copy(dst_hbm.at[r0 + 1], didx1)
        pltpu.async_copy(x_hbm.at[sidx1], rows1, sem1)
        pltpu.make_async_copy(x_hbm.at[sidx0], rows0, sem0).wait()
        pltpu.sync_copy(rows0, agg_s.at[didx0], add=True)

        @pl.when(jj < 39)
        def _():
            pltpu.sync_copy(src_hbm.at[r0 + 2], sidx0)
            pltpu.sync_copy(dst_hbm.at[r0 + 2], didx0)
            pltpu.async_copy(x_hbm.at[sidx0], rows0, sem0)
        pltpu.make_async_copy(x_hbm.at[sidx1], rows1, sem1).wait()
        pltpu.sync_copy(rows1, agg_s.at[didx1], add=True)
        return 0
    lax.fori_loop(0, 40, _pair, 0)
    plsc.subcore_barrier()

    pltpu.sync_copy(agg_s.at[pl.ds(pl.multiple_of(s * 640, 640), 640)],
                    out_hbm.at[c, pl.ds(pl.multiple_of(s * 640, 640), 640)])


def _tc_body(x_ref, agg_ref, invd_ref, batch_ref, c2_ref, ga_ref,
             wl1, wr1, wl2, wr2, w1g, w1a, w2p, bl1, bl2, b1r, b2r,
             out_ref, acc_c, acc_q, cnt):
    i = pl.program_id(0)

    @pl.when(i == 0)
    def _():
        acc_c[...] = jnp.zeros_like(acc_c)
        acc_q[...] = jnp.zeros_like(acc_q)
        cnt[...] = jnp.zeros_like(cnt)

    f32 = jnp.float32
    dot = functools.partial(lax.dot_general, preferred_element_type=f32)

    aggsum = agg_ref[0] + agg_ref[1]
    invd = invd_ref[0, 0, :]
    h1 = jnp.maximum(
        dot(aggsum * invd[:, None], wl1[...], (((1,), (1,)), ((), ())))
        + bl1[...]
        + dot(x_ref[...], wr1[...], (((1,), (1,)), ((), ()))),
        0.0)

    cblk = c2_ref[0] + c2_ref[1]
    acc_c[...] += dot(cblk, h1, (((1,), (0,)), ((), ())))

    b = batch_ref[0, 0, :]
    q = (lax.broadcasted_iota(jnp.int32, (_G, 512), 0) == b[None, :]).astype(f32)
    acc_q[...] += dot(q, h1, (((1,), (0,)), ((), ())))
    cnt[...] += jnp.broadcast_to(jnp.sum(q, axis=1)[:, None], (_G, 128))

    @pl.when(i == pl.num_programs(0) - 1)
    def _():
        cvec = cnt[...][:, :1]
        inv_cnt = 1.0 / jnp.maximum(cvec, 1.0)
        nonz = (cvec > 0.0).astype(f32)
        g2 = (dot(acc_c[...] * inv_cnt, wl2[...], (((1,), (1,)), ((), ())))
              + bl2[...] * nonz
              + dot(acc_q[...] * inv_cnt, wr2[...], (((1,), (1,)), ((), ()))))
        zz = jnp.maximum(
            dot(g2, w1g[...], (((1,), (1,)), ((), ())))
            + dot(ga_ref[...], w1a[...], (((1,), (1,)), ((), ())))
            + b1r[...], 0.0)
        out_ref[...] = dot(zz, w2p[...], (((1,), (1,)), ((), ()))) + b2r[...]


def _tc_dense(x_p, aggp, invd3, batch3, c2, ga_p,
              wl1, wr1, wl2, wr2, w1g, w1a, w2p, bl1, bl2, b1r, b2r):
    nblk = _NPAD // 512
    full = lambda shape: pl.BlockSpec(shape, lambda i: tuple(0 for _ in shape))
    return pl.pallas_call(
        _tc_body,
        grid=(nblk,),
        in_specs=[
            pl.BlockSpec((512, _F), lambda i: (i, 0)),
            pl.BlockSpec((2, 512, _F), lambda i: (0, i, 0)),
            pl.BlockSpec((1, 1, 512), lambda i: (i, 0, 0)),
            pl.BlockSpec((1, 1, 512), lambda i: (i, 0, 0)),
            pl.BlockSpec((2, _G, 512), lambda i: (0, 0, i)),
            full((_G, 128)),
            full((_H, _F)), full((_H, _F)), full((_H, _H)), full((_H, _H)),
            full((_H, _H)), full((_H, 128)), full((128, _H)),
            full((1, _H)), full((1, _H)), full((1, _H)), full((1, 128)),
        ],
        out_specs=pl.BlockSpec((_G, 128), lambda i: (0, 0)),
        out_shape=jax.ShapeDtypeStruct((_G, 128), jnp.float32),
        scratch_shapes=[
            pltpu.VMEM((_G, _H), jnp.float32),
            pltpu.VMEM((_G, _H), jnp.float32),
            pltpu.VMEM((_G, 128), jnp.float32),
        ],
    )(x_p, aggp, invd3, batch3, c2, ga_p,
      wl1, wr1, wl2, wr2, w1g, w1a, w2p, bl1, bl2, b1r, b2r)


def kernel(x, edge_index, batch, graph_attr,
           W_l1, b_l1, W_r1, W_l2, b_l2, W_r2, W1, b1, W2, b2):
    f32 = jnp.float32
    src = edge_index[0]
    dst = edge_index[1]
    epad = _EPAD - _E
    pad_iota = jnp.arange(epad, dtype=jnp.int32)
    src_p = jnp.concatenate([src, pad_iota % _N]).reshape(_EPAD // _CHUNK, _CHUNK)
    dst_p = jnp.concatenate([dst, _N + pad_iota % (_NPAD - _N)]).reshape(
        _EPAD // _CHUNK, _CHUNK)
    batch_p = jnp.concatenate([batch, jnp.full((_NPAD - _N,), _G, jnp.int32)])
    x_p = jnp.concatenate([x, jnp.zeros((_NPAD - _N, _F), f32)], axis=0)
    ga_p = jnp.concatenate([graph_attr, jnp.zeros((_G, 128 - _A), f32)], axis=1)

    inv_deg, c2_flat = _make_sc_degree_and_c()(src_p, dst_p, batch_p)
    aggp = _make_sc_aggregate()(src_p, dst_p, x)

    c2 = c2_flat.reshape(2, _G, _NPAD)
    invd3 = inv_deg.reshape(_NPAD // 512, 1, 512)
    batch3 = batch_p.reshape(_NPAD // 512, 1, 512)

    w1g = W1[:, :_H]
    w1a = jnp.concatenate([W1[:, _H:], jnp.zeros((_H, 128 - _A), f32)], axis=1)
    w2p = jnp.concatenate([W2, jnp.zeros((128 - _OUT, _H), f32)], axis=0)
    bl1 = b_l1.reshape(1, _H)
    bl2 = b_l2.reshape(1, _H)
    b1r = b1.reshape(1, _H)
    b2r = jnp.concatenate([b2, jnp.zeros((128 - _OUT,), f32)]).reshape(1, 128)

    out = _tc_dense(x_p, aggp, invd3, batch3, c2, ga_p,
                    W_l1, W_r1, W_l2, W_r2, w1g, w1a, w2p, bl1, bl2, b1r, b2r)
    return out[:, :_OUT]


# speedup vs baseline: 19.3043x; 1.0713x over previous
import functools

import jax
import jax.numpy as jnp
from jax import lax
from jax.experimental import pallas as pl
from jax.experimental.pallas import tpu as pltpu
from jax.experimental.pallas import tpu_sc as plsc

_N = 10000
_E = 320000
_F = 128
_H = 256
_G = 64
_A = 16
_OUT = 8

_NPAD = 10240
_EPAD = 327680
_CHUNK = 128
_CFLAT = 65 * _NPAD

@functools.cache
def _make_sc_degree_and_c():
  deco = functools.partial(
      pl.kernel,
      mesh=plsc.VectorSubcoreMesh(core_axis_name="c", subcore_axis_name="s"),
      compiler_params=pltpu.CompilerParams(needs_layout_passes=False),
      out_type=(
          jax.ShapeDtypeStruct((_NPAD,), jnp.float32),
          jax.ShapeDtypeStruct((2, _G * _NPAD), jnp.float32),
      ),
      scratch_types=[
          pltpu.VMEM((4160,), jnp.float32),
          pltpu.VMEM((_CHUNK,), jnp.float32),
          pltpu.VMEM((_CHUNK,), jnp.int32),
          pltpu.VMEM((160, _CHUNK), jnp.int32),
          pltpu.VMEM((80, _CHUNK), jnp.int32),
          pltpu.VMEM((80, _CHUNK), jnp.int32),
          pltpu.VMEM((_CHUNK,), jnp.float32),
          pltpu.VMEM((640,), jnp.float32),
          pltpu.VMEM((640,), jnp.float32),
          pltpu.VMEM((_NPAD,), jnp.int32),
          pltpu.VMEM((_NPAD,), jnp.float32),
          pltpu.VMEM((_CHUNK,), jnp.int32),
          pltpu.VMEM((_CHUNK,), jnp.float32),
          pltpu.VMEM_SHARED((_NPAD,), jnp.float32),
          pltpu.VMEM_SHARED((_NPAD,), jnp.float32),
          pltpu.VMEM_SHARED((_CFLAT,), jnp.float32),
          pltpu.SemaphoreType.DMA,
          pltpu.SemaphoreType.DMA,
          pltpu.SemaphoreType.DMA,
      ],
  )
  return deco(_sc_degree_and_c_body)


def _sc_degree_and_c_body(src_hbm, dst_hbm, batch_hbm, inv_out, c_out,
                          zbuf, ones_b, idxb, dbig, dc, sc_, updb, dslice,
                          islice, batch_v, inv_v, idxb1, updb1,
                          deg_s, inv_s, c_s, semd, semc0, semc1):
    c = lax.axis_index("c")
    s = lax.axis_index("s")

    def _fill(i, _):
        zbuf[pl.ds(pl.multiple_of(i * 16, 16), 16)] = jnp.zeros((16,), jnp.float32)
        return 0
    lax.fori_loop(0, 260, _fill, 0)
    for i in range(8):
        ones_b[pl.ds(i * 16, 16)] = jnp.ones((16,), jnp.float32)

    pltpu.sync_copy(zbuf.at[pl.ds(0, 640)],
                    deg_s.at[pl.ds(pl.multiple_of(s * 640, 640), 640)])
    cz_base = pl.multiple_of(s * 41600, 8)

    def _zero_c(j, _):
        pltpu.sync_copy(zbuf, c_s.at[pl.ds(pl.multiple_of(cz_base + j * 4160, 8), 4160)])
        return 0
    lax.fori_loop(0, 10, _zero_c, 0)
    plsc.subcore_barrier()

    pltpu.sync_copy(dst_hbm.at[pl.ds(pl.multiple_of(s * 160, 8), 160)], dbig)
    for k in range(8):
        pltpu.async_copy(ones_b, deg_s.at[dbig.at[k]], semd, add=True)

    def _dgrp(g, _):
        @pl.when(g < 19)
        def _():
            for k in range(8):
                pltpu.async_copy(ones_b, deg_s.at[dbig.at[(g + 1) * 8 + k]],
                                 semd, add=True)
        for k in range(8):
            pltpu.make_async_copy(ones_b, deg_s.at[dbig.at[g * 8 + k]], semd).wait()
        return 0
    lax.fori_loop(0, 20, _dgrp, 0)
    plsc.subcore_barrier()

    off = pl.multiple_of(s * 640, 640)
    pltpu.sync_copy(deg_s.at[pl.ds(off, 640)], dslice)

    def _inv(k, _):
        d16 = dslice[pl.ds(pl.multiple_of(k * 16, 16), 16)]
        islice[pl.ds(pl.multiple_of(k * 16, 16), 16)] = 1.0 / jnp.maximum(d16, 1.0)
        return 0
    lax.fori_loop(0, 40, _inv, 0)
    pltpu.sync_copy(islice, inv_s.at[pl.ds(off, 640)])

    @pl.when(c == 0)
    def _():
        pltpu.sync_copy(islice, inv_out.at[pl.ds(off, 640)])
    plsc.subcore_barrier()

    pltpu.sync_copy(batch_hbm, batch_v)
    pltpu.sync_copy(inv_s, inv_v)

    c_row0 = pl.multiple_of(c * 1280 + s * 80, 8)
    pltpu.sync_copy(src_hbm.at[pl.ds(c_row0, 80)], sc_)
    pltpu.sync_copy(dst_hbm.at[pl.ds(c_row0, 80)], dc)

    def _cbuild(j, idxr, updr):
        for k in range(_CHUNK // 16):
            d16 = dc[j, pl.ds(k * 16, 16)]
            s16 = sc_[j, pl.ds(k * 16, 16)]
            t16 = plsc.load_gather(batch_v, [d16])
            w16 = plsc.load_gather(inv_v, [d16])
            idxr[pl.ds(k * 16, 16)] = t16 * _NPAD + s16
            updr[pl.ds(k * 16, 16)] = w16

    _cbuild(0, idxb, updb)
    pltpu.async_copy(updb, c_s.at[idxb], semc0, add=True)

    def _cgrp(jj, _):
        c0 = jj * 2
        _cbuild(c0 + 1, idxb1, updb1)
        pltpu.async_copy(updb1, c_s.at[idxb1], semc1, add=True)
        pltpu.make_async_copy(updb, c_s.at[idxb], semc0).wait()

        @pl.when(jj < 39)
        def _():
            _cbuild(c0 + 2, idxb, updb)
            pltpu.async_copy(updb, c_s.at[idxb], semc0, add=True)
        pltpu.make_async_copy(updb1, c_s.at[idxb1], semc1).wait()
        return 0
    lax.fori_loop(0, 40, _cgrp, 0)
    plsc.subcore_barrier()

    out_off = pl.multiple_of(s * 40960, 8)
    pltpu.sync_copy(c_s.at[pl.ds(out_off, 40960)], c_out.at[c, pl.ds(out_off, 40960)])


@functools.cache
def _make_sc_aggregate():
  deco = functools.partial(
      pl.kernel,
      mesh=plsc.VectorSubcoreMesh(core_axis_name="c", subcore_axis_name="s"),
      compiler_params=pltpu.CompilerParams(needs_layout_passes=False),
      out_type=jax.ShapeDtypeStruct((2, _NPAD, _F), jnp.float32),
      scratch_types=[
          pltpu.VMEM((_CHUNK,), jnp.int32),
          pltpu.VMEM((_CHUNK,), jnp.int32),
          pltpu.VMEM((_CHUNK,), jnp.int32),
          pltpu.VMEM((_CHUNK,), jnp.int32),
          pltpu.VMEM((_CHUNK, _F), jnp.float32),
          pltpu.VMEM((_CHUNK, _F), jnp.float32),
          pltpu.VMEM_SHARED((_NPAD, _F), jnp.float32),
          pltpu.SemaphoreType.DMA,
          pltpu.SemaphoreType.DMA,
      ],
  )
  return deco(_sc_aggregate_body)


def _sc_aggregate_body(src_hbm, dst_hbm, x_hbm, out_hbm,
                       sidx0, didx0, sidx1, didx1, rows0, rows1,
                       agg_s, sem0, sem1):
    c = lax.axis_index("c")
    s = lax.axis_index("s")

    def _zero(k, _):
        i = k // 8
        j = k % 8
        rows0[i, pl.ds(pl.multiple_of(j * 16, 16), 16)] = jnp.zeros((16,), jnp.float32)
        return 0
    lax.fori_loop(0, _CHUNK * 8, _zero, 0)

    row_base = pl.multiple_of(s * 640, 128)

    def _zero_agg(j, _):
        pltpu.sync_copy(rows0, agg_s.at[pl.ds(pl.multiple_of(row_base + j * _CHUNK, _CHUNK), _CHUNK)])
        return 0
    lax.fori_loop(0, 5, _zero_agg, 0)
    plsc.subcore_barrier()

    row0 = c * 1280 + s * 80

    pltpu.sync_copy(src_hbm.at[row0], sidx0)
    pltpu.sync_copy(dst_hbm.at[row0], didx0)
    pltpu.async_copy(x_hbm.at[sidx0], rows0, sem0)

    def _pair(jj, _):
        r0 = row0 + jj * 2
        pltpu.sync_copy(src_hbm.at[r0 + 1], sidx1)
        pltpu.sync_copy(dst_hbm.at[r0 + 1], didx1)
        pltpu.async_copy(x_hbm.at[sidx1], rows1, sem1)
        pltpu.make_async_copy(x_hbm.at[sidx0], rows0, sem0).wait()
        pltpu.sync_copy(rows0, agg_s.at[didx0], add=True)

        @pl.when(jj < 39)
        def _():
            pltpu.sync_copy(src_hbm.at[r0 + 2], sidx0)
            pltpu.sync_copy(dst_hbm.at[r0 + 2], didx0)
            pltpu.async_copy(x_hbm.at[sidx0], rows0, sem0)
        pltpu.make_async_copy(x_hbm.at[sidx1], rows1, sem1).wait()
        pltpu.sync_copy(rows1, agg_s.at[didx1], add=True)
        return 0
    lax.fori_loop(0, 40, _pair, 0)
    plsc.subcore_barrier()

    pltpu.sync_copy(agg_s.at[pl.ds(pl.multiple_of(s * 640, 640), 640)],
                    out_hbm.at[c, pl.ds(pl.multiple_of(s * 640, 640), 640)])


def _tc_body(x_ref, agg_ref, invd_ref, batch_ref, c2_ref, ga_ref,
             wl1, wr1, wl2, wr2, w1g, w1a, w2p, bl1, bl2, b1r, b2r,
             out_ref, acc_c, acc_q, cnt):
    i = pl.program_id(0)

    @pl.when(i == 0)
    def _():
        acc_c[...] = jnp.zeros_like(acc_c)
        acc_q[...] = jnp.zeros_like(acc_q)
        cnt[...] = jnp.zeros_like(cnt)

    f32 = jnp.float32
    dot = functools.partial(lax.dot_general, preferred_element_type=f32)

    aggsum = agg_ref[0] + agg_ref[1]
    invd = invd_ref[0, 0, :]
    h1 = jnp.maximum(
        dot(aggsum * invd[:, None], wl1[...], (((1,), (1,)), ((), ())))
        + bl1[...]
        + dot(x_ref[...], wr1[...], (((1,), (1,)), ((), ()))),
        0.0)

    cblk = c2_ref[0] + c2_ref[1]
    acc_c[...] += dot(cblk, h1, (((1,), (0,)), ((), ())))

    b = batch_ref[0, 0, :]
    q = (lax.broadcasted_iota(jnp.int32, (_G, 512), 0) == b[None, :]).astype(f32)
    acc_q[...] += dot(q, h1, (((1,), (0,)), ((), ())))
    cnt[...] += jnp.broadcast_to(jnp.sum(q, axis=1)[:, None], (_G, 128))

    @pl.when(i == pl.num_programs(0) - 1)
    def _():
        cvec = cnt[...][:, :1]
        inv_cnt = 1.0 / jnp.maximum(cvec, 1.0)
        nonz = (cvec > 0.0).astype(f32)
        g2 = (dot(acc_c[...] * inv_cnt, wl2[...], (((1,), (1,)), ((), ())))
              + bl2[...] * nonz
              + dot(acc_q[...] * inv_cnt, wr2[...], (((1,), (1,)), ((), ()))))
        zz = jnp.maximum(
            dot(g2, w1g[...], (((1,), (1,)), ((), ())))
            + dot(ga_ref[...], w1a[...], (((1,), (1,)), ((), ())))
            + b1r[...], 0.0)
        out_ref[...] = dot(zz, w2p[...], (((1,), (1,)), ((), ()))) + b2r[...]


def _tc_dense(x_p, aggp, invd3, batch3, c2, ga_p,
              wl1, wr1, wl2, wr2, w1g, w1a, w2p, bl1, bl2, b1r, b2r):
    nblk = _NPAD // 512
    full = lambda shape: pl.BlockSpec(shape, lambda i: tuple(0 for _ in shape))
    return pl.pallas_call(
        _tc_body,
        grid=(nblk,),
        in_specs=[
            pl.BlockSpec((512, _F), lambda i: (i, 0)),
            pl.BlockSpec((2, 512, _F), lambda i: (0, i, 0)),
            pl.BlockSpec((1, 1, 512), lambda i: (i, 0, 0)),
            pl.BlockSpec((1, 1, 512), lambda i: (i, 0, 0)),
            pl.BlockSpec((2, _G, 512), lambda i: (0, 0, i)),
            full((_G, 128)),
            full((_H, _F)), full((_H, _F)), full((_H, _H)), full((_H, _H)),
            full((_H, _H)), full((_H, 128)), full((128, _H)),
            full((1, _H)), full((1, _H)), full((1, _H)), full((1, 128)),
        ],
        out_specs=pl.BlockSpec((_G, 128), lambda i: (0, 0)),
        out_shape=jax.ShapeDtypeStruct((_G, 128), jnp.float32),
        scratch_shapes=[
            pltpu.VMEM((_G, _H), jnp.float32),
            pltpu.VMEM((_G, _H), jnp.float32),
            pltpu.VMEM((_G, 128), jnp.float32),
        ],
    )(x_p, aggp, invd3, batch3, c2, ga_p,
      wl1, wr1, wl2, wr2, w1g, w1a, w2p, bl1, bl2, b1r, b2r)


def kernel(x, edge_index, batch, graph_attr,
           W_l1, b_l1, W_r1, W_l2, b_l2, W_r2, W1, b1, W2, b2):
    f32 = jnp.float32
    src = edge_index[0]
    dst = edge_index[1]
    epad = _EPAD - _E
    pad_iota = jnp.arange(epad, dtype=jnp.int32)
    src_p = jnp.concatenate([src, pad_iota % _N]).reshape(_EPAD // _CHUNK, _CHUNK)
    dst_p = jnp.concatenate([dst, _N + pad_iota % (_NPAD - _N)]).reshape(
        _EPAD // _CHUNK, _CHUNK)
    batch_p = jnp.concatenate([batch, jnp.full((_NPAD - _N,), _G, jnp.int32)])
    x_p = jnp.concatenate([x, jnp.zeros((_NPAD - _N, _F), f32)], axis=0)
    ga_p = jnp.concatenate([graph_attr, jnp.zeros((_G, 128 - _A), f32)], axis=1)

    inv_deg, c2_flat = _make_sc_degree_and_c()(src_p, dst_p, batch_p)
    aggp = _make_sc_aggregate()(src_p, dst_p, x)

    c2 = c2_flat.reshape(2, _G, _NPAD)
    invd3 = inv_deg.reshape(_NPAD // 512, 1, 512)
    batch3 = batch_p.reshape(_NPAD // 512, 1, 512)

    w1g = W1[:, :_H]
    w1a = jnp.concatenate([W1[:, _H:], jnp.zeros((_H, 128 - _A), f32)], axis=1)
    w2p = jnp.concatenate([W2, jnp.zeros((128 - _OUT, _H), f32)], axis=0)
    bl1 = b_l1.reshape(1, _H)
    bl2 = b_l2.reshape(1, _H)
    b1r = b1.reshape(1, _H)
    b2r = jnp.concatenate([b2, jnp.zeros((128 - _OUT,), f32)]).reshape(1, 128)

    out = _tc_dense(x_p, aggp, invd3, batch3, c2, ga_p,
                    W_l1, W_r1, W_l2, W_r2, w1g, w1a, w2p, bl1, bl2, b1r, b2r)
    return out[:, :_OUT]
